# baseline (speedup 1.0000x reference)
import functools

import jax
import jax.numpy as jnp
from jax import lax
from jax.experimental import pallas as pl
from jax.experimental.pallas import tpu as pltpu
from jax.experimental.pallas import tpu_sc as plsc

_NTYPE = 9
_N = 50000
_NPAD = 50176
_E = 300000
_EPAD = 307200
_NC = 2
_NS = 16
_ECH = _EPAD // (_NC * _NS)
_PT = _NPAD // _NS

_MESH = plsc.VectorSubcoreMesh(
    core_axis_name="c", subcore_axis_name="s", num_cores=_NC, num_subcores=_NS)


def _fill(buf, nwords, value):
    @pl.loop(0, nwords // 16)
    def _(i):
        buf[pl.ds(i * 16, 16)] = jnp.full((16,), value, buf.dtype)


def _deg_body(sidx_hbm, ridx_hbm, w_hbm, deg_hbm, sidx_v, ridx_v, w_v, zbuf, deg_sp):
    cid = lax.axis_index("c")
    sid = lax.axis_index("s")
    base = (sid * _NC + cid) * _ECH
    pltpu.sync_copy(sidx_hbm.at[pl.ds(base, _ECH)], sidx_v)
    pltpu.sync_copy(ridx_hbm.at[pl.ds(base, _ECH)], ridx_v)
    pltpu.sync_copy(w_hbm.at[pl.ds(base, _ECH)], w_v)
    _fill(zbuf, _PT, 0.0)
    pltpu.sync_copy(zbuf, deg_sp.at[pl.ds(sid * _PT, _PT)])
    plsc.subcore_barrier()
    pltpu.sync_copy(w_v, deg_sp.at[sidx_v], add=True)
    pltpu.sync_copy(w_v, deg_sp.at[ridx_v], add=True)
    plsc.subcore_barrier()
    pltpu.sync_copy(deg_sp.at[pl.ds(sid * _PT, _PT)], zbuf)
    pltpu.sync_copy(zbuf, deg_hbm.at[pl.ds(cid * _NPAD + sid * _PT, _PT)])


_deg_call = pl.kernel(
    _deg_body,
    out_type=jax.ShapeDtypeStruct((_NC * _NPAD,), jnp.float32),
    mesh=_MESH,
    scratch_types=[
        pltpu.VMEM((_ECH,), jnp.int32),
        pltpu.VMEM((_ECH,), jnp.int32),
        pltpu.VMEM((_ECH,), jnp.float32),
        pltpu.VMEM((_PT,), jnp.float32),
        pltpu.VMEM_SHARED((_NPAD,), jnp.float32),
    ],
)


def _msg_body(sidx_hbm, ridx_hbm, w_hbm, z0_hbm, z1_hbm, z2_hbm, g_hbm,
              sidx_v, ridx_v, w_v, zg0, zg1, zg2, zg3, zg4, zg5, zbuf,
              g0, g1, g2, zt0, zt1, zt2, s0, s1, s2, s3, s4, s5):
    cid = lax.axis_index("c")
    sid = lax.axis_index("s")
    base = (sid * _NC + cid) * _ECH
    pltpu.sync_copy(sidx_hbm.at[pl.ds(base, _ECH)], sidx_v)
    pltpu.sync_copy(ridx_hbm.at[pl.ds(base, _ECH)], ridx_v)
    pltpu.sync_copy(w_hbm.at[pl.ds(base, _ECH)], w_v)
    zgs = (zg0, zg1, zg2, zg3, zg4, zg5)
    sems = (s0, s1, s2, s3, s4, s5)
    gtabs = (g0, g1, g2)
    ztabs = (zt0, zt1, zt2)
    nsl = pl.ds(sid * _PT, _PT)
    hbm_gathers = []
    for c, z_hbm in enumerate((z0_hbm, z1_hbm, z2_hbm)):
        hbm_gathers.append(
            pltpu.async_copy(z_hbm.at[sidx_v], zgs[2 * c + 1], sems[2 * c + 1]))
    for z_hbm, ztab in zip((z0_hbm, z1_hbm, z2_hbm), ztabs):
        pltpu.sync_copy(z_hbm.at[nsl], zbuf)
        pltpu.sync_copy(zbuf, ztab.at[nsl])
    _fill(zbuf, _PT, 0.0)
    for gtab in gtabs:
        pltpu.sync_copy(zbuf, gtab.at[nsl])
    plsc.subcore_barrier()
    gathers = []
    for c, ztab in enumerate(ztabs):
        gathers.append(pltpu.async_copy(ztab.at[ridx_v], zgs[2 * c], sems[2 * c]))
        gathers.append(hbm_gathers[c])
    scatters = []
    for j in range(6):
        gathers[j].wait()
        zg = zgs[j]

        @pl.loop(0, _ECH // 16)
        def _(i):
            zg[pl.ds(i * 16, 16)] = (
                w_v[pl.ds(i * 16, 16)] * zg[pl.ds(i * 16, 16)])

        dst = sidx_v if j % 2 == 0 else ridx_v
        scatters.append(
            pltpu.async_copy(zg, gtabs[j // 2].at[dst], sems[j], add=True))
    for sc in scatters:
        sc.wait()
    plsc.subcore_barrier()
    for c, gtab in enumerate(gtabs):
        pltpu.sync_copy(gtab.at[pl.ds(sid * _PT, _PT)], zbuf)
        pltpu.sync_copy(zbuf, g_hbm.at[pl.ds((cid * 3 + c) * _NPAD + sid * _PT, _PT)])


_msg_call = pl.kernel(
    _msg_body,
    out_type=jax.ShapeDtypeStruct((_NC * 3 * _NPAD,), jnp.float32),
    mesh=_MESH,
    scratch_types=(
        [pltpu.VMEM((_ECH,), jnp.int32)] * 2
        + [pltpu.VMEM((_ECH,), jnp.float32)] * 7
        + [pltpu.VMEM((_PT,), jnp.float32)]
        + [pltpu.VMEM_SHARED((_NPAD,), jnp.float32)] * 6
        + [pltpu.SemaphoreType.DMA] * 6
    ),
)


def _col_mask(x):
    cols = lax.broadcasted_iota(jnp.int32, x.shape, 1)
    return jnp.where(cols < _N, x, 0.0)


def _prep_body(wpT_ref, pwpT_ref, ntT_ref, deg_ref, W1_ref, b1_ref, z_ref, dis_ref):
    vel = wpT_ref[...] - pwpT_ref[...]
    rows = lax.broadcasted_iota(jnp.int32, (_NTYPE, _NPAD), 0)
    onehot = (rows == jnp.broadcast_to(ntT_ref[...], (_NTYPE, _NPAD))).astype(jnp.float32)
    nfT = jnp.concatenate([vel, onehot], axis=0)
    y = lax.dot_general(W1_ref[...], nfT, (((1,), (0,)), ((), ())),
                        preferred_element_type=jnp.float32)
    deg = deg_ref[0:1, :] + deg_ref[1:2, :] + 2.0
    dis = lax.rsqrt(deg)
    z_ref[...] = _col_mask(dis * (y + b1_ref[...]))
    dis_ref[...] = dis


def _prep_call(wpT, pwpT, ntT, deg2, W1, b1col):
    return pl.pallas_call(
        _prep_body,
        out_shape=[jax.ShapeDtypeStruct((3, _NPAD), jnp.float32),
                   jax.ShapeDtypeStruct((1, _NPAD), jnp.float32)],
    )(wpT, pwpT, ntT, deg2, W1, b1col)


def _mid_body(g_ref, dis_ref, z_ref, W2_ref, b2_ref, z2_ref):
    dis = dis_ref[...]
    h = dis * g_ref[...] + 2.0 * dis * z_ref[...]
    y2 = lax.dot_general(W2_ref[...], h, (((1,), (0,)), ((), ())),
                         preferred_element_type=jnp.float32)
    z2_ref[...] = _col_mask(dis * (y2 + b2_ref[...]))


def _mid_call(gT, dis, z, W2, b2col):
    return pl.pallas_call(
        _mid_body,
        out_shape=jax.ShapeDtypeStruct((3, _NPAD), jnp.float32),
    )(gT, dis, z, W2, b2col)


def _fin_body(g_ref, dis_ref, z_ref, out_ref):
    dis = dis_ref[...]
    out_ref[...] = dis * g_ref[...] + 2.0 * dis * z_ref[...]


def _fin_call(gT, dis, z):
    return pl.pallas_call(
        _fin_body,
        out_shape=jax.ShapeDtypeStruct((3, _NPAD), jnp.float32),
    )(gT, dis, z)


def _from_planes(g6):
    g = g6.reshape(_NC * 3, _NPAD)
    return g[0:3] + g[3:6]


def kernel(world_pos, prev_world_pos, node_type, cells, mesh_pos, is_training, W1, b1, W2, b2):
    ct = cells.T.astype(jnp.uint32)
    a, b, c = ct[0], ct[1], ct[2]

    def _pack(x, y):
        return (jnp.maximum(x, y) << 16) | jnp.minimum(x, y)

    k1 = _pack(a, b)
    keyp = jnp.concatenate(
        [k1, _pack(b, c), _pack(c, a),
         jnp.broadcast_to(k1[0], (_EPAD - _E,))])
    sk = lax.sort(keyp, is_stable=False)
    prev = jnp.concatenate([sk[:1] ^ jnp.uint32(1), sk[:-1]])
    w = (sk != prev).astype(jnp.float32)
    sidx = (sk >> 16).astype(jnp.int32)
    ridx = (sk & jnp.uint32(0xFFFF)).astype(jnp.int32)

    pad = ((0, _NPAD - _N), (0, 0))
    wpT = jnp.pad(world_pos, pad).T
    pwpT = jnp.pad(prev_world_pos, pad).T
    ntT = jnp.pad(node_type, pad).T
    b1col = b1[:, None]
    b2col = b2[:, None]

    deg2 = _deg_call(sidx, ridx, w).reshape(_NC, _NPAD)
    z1, dis = _prep_call(wpT, pwpT, ntT, deg2, W1, b1col)

    g1 = _from_planes(_msg_call(sidx, ridx, w, z1[0], z1[1], z1[2]))
    z2 = _mid_call(g1, dis, z1, W2, b2col)

    g2 = _from_planes(_msg_call(sidx, ridx, w, z2[0], z2[1], z2[2]))
    h2T = _fin_call(g2, dis, z2)

    h = h2T[:, :_N].T
    return jnp.where(is_training != 0, h, 2.0 * world_pos + h - prev_world_pos)


# speedup vs baseline: 1.1582x; 1.1582x over previous
import functools

import jax
import jax.numpy as jnp
from jax import lax
from jax.experimental import pallas as pl
from jax.experimental.pallas import tpu as pltpu
from jax.experimental.pallas import tpu_sc as plsc

_NTYPE = 9
_N = 50000
_NPAD = 50176
_E = 300000
_EPAD = 307200
_NC = 2
_NS = 16
_ECH = _EPAD // (_NC * _NS)
_PT = _NPAD // _NS

_MESH = plsc.VectorSubcoreMesh(
    core_axis_name="c", subcore_axis_name="s", num_cores=_NC, num_subcores=_NS)


def _fill(buf, nwords, value):
    @pl.loop(0, nwords // 16)
    def _(i):
        buf[pl.ds(i * 16, 16)] = jnp.full((16,), value, buf.dtype)


def _deg_body(sidx_hbm, ridx_hbm, w_hbm, deg_hbm, sidx_v, ridx_v, w_v, zbuf,
              deg_sp, dsem_a, dsem_b):
    cid = lax.axis_index("c")
    sid = lax.axis_index("s")
    base = (sid * _NC + cid) * _ECH
    pltpu.sync_copy(sidx_hbm.at[pl.ds(base, _ECH)], sidx_v)
    pltpu.sync_copy(ridx_hbm.at[pl.ds(base, _ECH)], ridx_v)
    pltpu.sync_copy(w_hbm.at[pl.ds(base, _ECH)], w_v)
    _fill(zbuf, _PT, 0.0)
    pltpu.sync_copy(zbuf, deg_sp.at[pl.ds(sid * _PT, _PT)])
    plsc.subcore_barrier()
    sc_a = pltpu.async_copy(w_v, deg_sp.at[sidx_v], dsem_a, add=True)
    sc_b = pltpu.async_copy(w_v, deg_sp.at[ridx_v], dsem_b, add=True)
    sc_a.wait()
    sc_b.wait()
    plsc.subcore_barrier()
    pltpu.sync_copy(deg_sp.at[pl.ds(sid * _PT, _PT)], zbuf)
    pltpu.sync_copy(zbuf, deg_hbm.at[pl.ds(cid * _NPAD + sid * _PT, _PT)])


_deg_call = pl.kernel(
    _deg_body,
    out_type=jax.ShapeDtypeStruct((_NC * _NPAD,), jnp.float32),
    mesh=_MESH,
    scratch_types=[
        pltpu.VMEM((_ECH,), jnp.int32),
        pltpu.VMEM((_ECH,), jnp.int32),
        pltpu.VMEM((_ECH,), jnp.float32),
        pltpu.VMEM((_PT,), jnp.float32),
        pltpu.VMEM_SHARED((_NPAD,), jnp.float32),
        pltpu.SemaphoreType.DMA,
        pltpu.SemaphoreType.DMA,
    ],
)


def _msg_body(sidx_hbm, ridx_hbm, w_hbm, z0_hbm, z1_hbm, z2_hbm, g_hbm,
              sidx_v, ridx_v, w_v, zg0, zg1, zg2, zg3, zg4, zg5, zbuf,
              g0, g1, g2, zt0, zt1, zt2, s0, s1, s2, s3, s4, s5):
    cid = lax.axis_index("c")
    sid = lax.axis_index("s")
    base = (sid * _NC + cid) * _ECH
    pltpu.sync_copy(sidx_hbm.at[pl.ds(base, _ECH)], sidx_v)
    pltpu.sync_copy(ridx_hbm.at[pl.ds(base, _ECH)], ridx_v)
    pltpu.sync_copy(w_hbm.at[pl.ds(base, _ECH)], w_v)
    zgs = (zg0, zg1, zg2, zg3, zg4, zg5)
    sems = (s0, s1, s2, s3, s4, s5)
    gtabs = (g0, g1, g2)
    ztabs = (zt0, zt1, zt2)
    nsl = pl.ds(sid * _PT, _PT)
    for z_hbm, ztab in zip((z0_hbm, z1_hbm, z2_hbm), ztabs):
        pltpu.sync_copy(z_hbm.at[nsl], zbuf)
        pltpu.sync_copy(zbuf, ztab.at[nsl])
    _fill(zbuf, _PT, 0.0)
    for gtab in gtabs:
        pltpu.sync_copy(zbuf, gtab.at[nsl])
    plsc.subcore_barrier()
    gathers = []
    for c, ztab in enumerate(ztabs):
        gathers.append(pltpu.async_copy(ztab.at[ridx_v], zgs[2 * c], sems[2 * c]))
        gathers.append(pltpu.async_copy(ztab.at[sidx_v], zgs[2 * c + 1], sems[2 * c + 1]))
    scatters = []
    for j in range(6):
        gathers[j].wait()
        zg = zgs[j]

        @pl.loop(0, _ECH // 16)
        def _(i):
            zg[pl.ds(i * 16, 16)] = (
                w_v[pl.ds(i * 16, 16)] * zg[pl.ds(i * 16, 16)])

        dst = sidx_v if j % 2 == 0 else ridx_v
        scatters.append(
            pltpu.async_copy(zg, gtabs[j // 2].at[dst], sems[j], add=True))
    for sc in scatters:
        sc.wait()
    plsc.subcore_barrier()
    for c, gtab in enumerate(gtabs):
        pltpu.sync_copy(gtab.at[pl.ds(sid * _PT, _PT)], zbuf)
        pltpu.sync_copy(zbuf, g_hbm.at[pl.ds((cid * 3 + c) * _NPAD + sid * _PT, _PT)])


_msg_call = pl.kernel(
    _msg_body,
    out_type=jax.ShapeDtypeStruct((_NC * 3 * _NPAD,), jnp.float32),
    mesh=_MESH,
    scratch_types=(
        [pltpu.VMEM((_ECH,), jnp.int32)] * 2
        + [pltpu.VMEM((_ECH,), jnp.float32)] * 7
        + [pltpu.VMEM((_PT,), jnp.float32)]
        + [pltpu.VMEM_SHARED((_NPAD,), jnp.float32)] * 6
        + [pltpu.SemaphoreType.DMA] * 6
    ),
)


def _col_mask(x):
    cols = lax.broadcasted_iota(jnp.int32, x.shape, 1)
    return jnp.where(cols < _N, x, 0.0)


def _prep_body(wpT_ref, pwpT_ref, ntT_ref, deg_ref, W1_ref, b1_ref, z_ref, dis_ref):
    vel = wpT_ref[...] - pwpT_ref[...]
    rows = lax.broadcasted_iota(jnp.int32, (_NTYPE, _NPAD), 0)
    onehot = (rows == jnp.broadcast_to(ntT_ref[...], (_NTYPE, _NPAD))).astype(jnp.float32)
    nfT = jnp.concatenate([vel, onehot], axis=0)
    y = lax.dot_general(W1_ref[...], nfT, (((1,), (0,)), ((), ())),
                        preferred_element_type=jnp.float32)
    deg = deg_ref[0:1, :] + deg_ref[1:2, :] + 2.0
    dis = lax.rsqrt(deg)
    z_ref[...] = _col_mask(dis * (y + b1_ref[...]))
    dis_ref[...] = dis


def _prep_call(wpT, pwpT, ntT, deg2, W1, b1col):
    return pl.pallas_call(
        _prep_body,
        out_shape=[jax.ShapeDtypeStruct((3, _NPAD), jnp.float32),
                   jax.ShapeDtypeStruct((1, _NPAD), jnp.float32)],
    )(wpT, pwpT, ntT, deg2, W1, b1col)


def _mid_body(g_ref, dis_ref, z_ref, W2_ref, b2_ref, z2_ref):
    dis = dis_ref[...]
    h = dis * g_ref[...] + 2.0 * dis * z_ref[...]
    y2 = lax.dot_general(W2_ref[...], h, (((1,), (0,)), ((), ())),
                         preferred_element_type=jnp.float32)
    z2_ref[...] = _col_mask(dis * (y2 + b2_ref[...]))


def _mid_call(gT, dis, z, W2, b2col):
    return pl.pallas_call(
        _mid_body,
        out_shape=jax.ShapeDtypeStruct((3, _NPAD), jnp.float32),
    )(gT, dis, z, W2, b2col)


def _fin_body(g_ref, dis_ref, z_ref, out_ref):
    dis = dis_ref[...]
    out_ref[...] = dis * g_ref[...] + 2.0 * dis * z_ref[...]


def _fin_call(gT, dis, z):
    return pl.pallas_call(
        _fin_body,
        out_shape=jax.ShapeDtypeStruct((3, _NPAD), jnp.float32),
    )(gT, dis, z)


def _from_planes(g6):
    g = g6.reshape(_NC * 3, _NPAD)
    return g[0:3] + g[3:6]


def kernel(world_pos, prev_world_pos, node_type, cells, mesh_pos, is_training, W1, b1, W2, b2):
    ct = cells.T.astype(jnp.uint32)
    a, b, c = ct[0], ct[1], ct[2]

    def _pack(x, y):
        return (jnp.maximum(x, y) << 16) | jnp.minimum(x, y)

    k1 = _pack(a, b)
    keyp = jnp.concatenate(
        [k1, _pack(b, c), _pack(c, a),
         jnp.broadcast_to(k1[0], (_EPAD - _E,))])
    sk = lax.sort(keyp, is_stable=False)
    prev = jnp.concatenate([sk[:1] ^ jnp.uint32(1), sk[:-1]])
    w = (sk != prev).astype(jnp.float32)
    sidx = (sk >> 16).astype(jnp.int32)
    ridx = (sk & jnp.uint32(0xFFFF)).astype(jnp.int32)

    pad = ((0, _NPAD - _N), (0, 0))
    wpT = jnp.pad(world_pos, pad).T
    pwpT = jnp.pad(prev_world_pos, pad).T
    ntT = jnp.pad(node_type, pad).T
    b1col = b1[:, None]
    b2col = b2[:, None]

    deg2 = _deg_call(sidx, ridx, w).reshape(_NC, _NPAD)
    z1, dis = _prep_call(wpT, pwpT, ntT, deg2, W1, b1col)

    g1 = _from_planes(_msg_call(sidx, ridx, w, z1[0], z1[1], z1[2]))
    z2 = _mid_call(g1, dis, z1, W2, b2col)

    g2 = _from_planes(_msg_call(sidx, ridx, w, z2[0], z2[1], z2[2]))
    h2T = _fin_call(g2, dis, z2)

    h = h2T[:, :_N].T
    return jnp.where(is_training != 0, h, 2.0 * world_pos + h - prev_world_pos)


# speedup vs baseline: 1.1587x; 1.0005x over previous
import jax
import jax.numpy as jnp
from jax import lax
from jax.experimental import pallas as pl
from jax.experimental.pallas import tpu as pltpu
from jax.experimental.pallas import tpu_sc as plsc

_NTYPE = 9
_N = 50000
_NPAD = 50176
_E = 300000
_EPAD = 307200
_NC = 2
_NS = 16
_ECH = _EPAD // (_NC * _NS)
_PT = _NPAD // _NS

_MESH = plsc.VectorSubcoreMesh(
    core_axis_name="c", subcore_axis_name="s", num_cores=_NC, num_subcores=_NS)


def _fill(buf, nwords, value):
    @pl.loop(0, nwords // 16)
    def _(i):
        buf[pl.ds(i * 16, 16)] = jnp.full((16,), value, buf.dtype)


def _deg_body(sidx_hbm, ridx_hbm, w_hbm, deg_hbm, sidx_v, ridx_v, w_v, zbuf,
              deg_sp, dsem_a, dsem_b):
    cid = lax.axis_index("c")
    sid = lax.axis_index("s")
    base = (sid * _NC + cid) * _ECH
    pltpu.sync_copy(sidx_hbm.at[pl.ds(base, _ECH)], sidx_v)
    pltpu.sync_copy(ridx_hbm.at[pl.ds(base, _ECH)], ridx_v)
    pltpu.sync_copy(w_hbm.at[pl.ds(base, _ECH)], w_v)
    _fill(zbuf, _PT, 0.0)
    pltpu.sync_copy(zbuf, deg_sp.at[pl.ds(sid * _PT, _PT)])
    plsc.subcore_barrier()
    sc_a = pltpu.async_copy(w_v, deg_sp.at[sidx_v], dsem_a, add=True)
    sc_b = pltpu.async_copy(w_v, deg_sp.at[ridx_v], dsem_b, add=True)
    sc_a.wait()
    sc_b.wait()
    plsc.subcore_barrier()
    pltpu.sync_copy(deg_sp.at[pl.ds(sid * _PT, _PT)], zbuf)
    pltpu.sync_copy(zbuf, deg_hbm.at[pl.ds(cid * _NPAD + sid * _PT, _PT)])


_deg_call = pl.kernel(
    _deg_body,
    out_type=jax.ShapeDtypeStruct((_NC * _NPAD,), jnp.float32),
    mesh=_MESH,
    scratch_types=[
        pltpu.VMEM((_ECH,), jnp.int32),
        pltpu.VMEM((_ECH,), jnp.int32),
        pltpu.VMEM((_ECH,), jnp.float32),
        pltpu.VMEM((_PT,), jnp.float32),
        pltpu.VMEM_SHARED((_NPAD,), jnp.float32),
        pltpu.SemaphoreType.DMA,
        pltpu.SemaphoreType.DMA,
    ],
)


def _msg_body(sidx_hbm, ridx_hbm, w_hbm, z0_hbm, z1_hbm, z2_hbm, g_hbm,
              sidx_v, ridx_v, w_v, zg0, zg1, zg2, zg3, zg4, zg5, zbuf,
              g0, g1, g2, zt0, zt1, zt2, s0, s1, s2, s3, s4, s5):
    cid = lax.axis_index("c")
    sid = lax.axis_index("s")
    base = (sid * _NC + cid) * _ECH
    pltpu.sync_copy(sidx_hbm.at[pl.ds(base, _ECH)], sidx_v)
    pltpu.sync_copy(ridx_hbm.at[pl.ds(base, _ECH)], ridx_v)
    pltpu.sync_copy(w_hbm.at[pl.ds(base, _ECH)], w_v)
    zgs = (zg0, zg1, zg2, zg3, zg4, zg5)
    sems = (s0, s1, s2, s3, s4, s5)
    gtabs = (g0, g1, g2)
    ztabs = (zt0, zt1, zt2)
    nsl = pl.ds(sid * _PT, _PT)
    for z_hbm, ztab in zip((z0_hbm, z1_hbm, z2_hbm), ztabs):
        pltpu.sync_copy(z_hbm.at[nsl], zbuf)
        pltpu.sync_copy(zbuf, ztab.at[nsl])
    _fill(zbuf, _PT, 0.0)
    for gtab in gtabs:
        pltpu.sync_copy(zbuf, gtab.at[nsl])
    plsc.subcore_barrier()
    gathers = []
    for c, ztab in enumerate(ztabs):
        gathers.append(pltpu.async_copy(ztab.at[ridx_v], zgs[2 * c], sems[2 * c]))
        gathers.append(pltpu.async_copy(ztab.at[sidx_v], zgs[2 * c + 1], sems[2 * c + 1]))
    scatters = []
    for j in range(6):
        gathers[j].wait()
        zg = zgs[j]

        @pl.loop(0, _ECH // 16)
        def _(i):
            zg[pl.ds(i * 16, 16)] = (
                w_v[pl.ds(i * 16, 16)] * zg[pl.ds(i * 16, 16)])

        dst = sidx_v if j % 2 == 0 else ridx_v
        scatters.append(
            pltpu.async_copy(zg, gtabs[j // 2].at[dst], sems[j], add=True))
    for sc in scatters:
        sc.wait()
    plsc.subcore_barrier()
    for c, gtab in enumerate(gtabs):
        pltpu.sync_copy(gtab.at[pl.ds(sid * _PT, _PT)], zbuf)
        pltpu.sync_copy(zbuf, g_hbm.at[pl.ds((cid * 3 + c) * _NPAD + sid * _PT, _PT)])


_msg_call = pl.kernel(
    _msg_body,
    out_type=jax.ShapeDtypeStruct((_NC * 3 * _NPAD,), jnp.float32),
    mesh=_MESH,
    scratch_types=(
        [pltpu.VMEM((_ECH,), jnp.int32)] * 2
        + [pltpu.VMEM((_ECH,), jnp.float32)] * 7
        + [pltpu.VMEM((_PT,), jnp.float32)]
        + [pltpu.VMEM_SHARED((_NPAD,), jnp.float32)] * 6
        + [pltpu.SemaphoreType.DMA] * 6
    ),
)


def _col_mask(x):
    cols = lax.broadcasted_iota(jnp.int32, x.shape, 1)
    return jnp.where(cols < _N, x, 0.0)


def _prep_body(wpT_ref, pwpT_ref, ntT_ref, deg_ref, W1_ref, b1_ref, z_ref, dis_ref):
    vel = wpT_ref[...] - pwpT_ref[...]
    rows = lax.broadcasted_iota(jnp.int32, (_NTYPE, _NPAD), 0)
    onehot = (rows == jnp.broadcast_to(ntT_ref[...], (_NTYPE, _NPAD))).astype(jnp.float32)
    nfT = jnp.concatenate([vel, onehot], axis=0)
    y = lax.dot_general(W1_ref[...], nfT, (((1,), (0,)), ((), ())),
                        preferred_element_type=jnp.float32)
    deg = deg_ref[0:1, :] + deg_ref[1:2, :] + 2.0
    dis = lax.rsqrt(deg)
    z_ref[...] = _col_mask(dis * (y + b1_ref[...]))
    dis_ref[...] = dis


def _prep_call(wpT, pwpT, ntT, deg2, W1, b1col):
    return pl.pallas_call(
        _prep_body,
        out_shape=[jax.ShapeDtypeStruct((3, _NPAD), jnp.float32),
                   jax.ShapeDtypeStruct((1, _NPAD), jnp.float32)],
    )(wpT, pwpT, ntT, deg2, W1, b1col)


def _mid_body(g_ref, dis_ref, z_ref, W2_ref, b2_ref, z2_ref):
    dis = dis_ref[...]
    h = dis * g_ref[...] + 2.0 * dis * z_ref[...]
    y2 = lax.dot_general(W2_ref[...], h, (((1,), (0,)), ((), ())),
                         preferred_element_type=jnp.float32)
    z2_ref[...] = _col_mask(dis * (y2 + b2_ref[...]))


def _mid_call(gT, dis, z, W2, b2col):
    return pl.pallas_call(
        _mid_body,
        out_shape=jax.ShapeDtypeStruct((3, _NPAD), jnp.float32),
    )(gT, dis, z, W2, b2col)


def _fin_body(g_ref, dis_ref, z_ref, out_ref):
    dis = dis_ref[...]
    out_ref[...] = dis * g_ref[...] + 2.0 * dis * z_ref[...]


def _fin_call(gT, dis, z):
    return pl.pallas_call(
        _fin_body,
        out_shape=jax.ShapeDtypeStruct((3, _NPAD), jnp.float32),
    )(gT, dis, z)


def _from_planes(g6):
    g = g6.reshape(_NC * 3, _NPAD)
    return g[0:3] + g[3:6]


def kernel(world_pos, prev_world_pos, node_type, cells, mesh_pos, is_training, W1, b1, W2, b2):
    ct = cells.T.astype(jnp.uint32)
    a, b, c = ct[0], ct[1], ct[2]

    def _pack(x, y):
        return (jnp.maximum(x, y) << 16) | jnp.minimum(x, y)

    k1 = _pack(a, b)
    keyp = jnp.concatenate(
        [k1, _pack(b, c), _pack(c, a),
         jnp.broadcast_to(k1[0], (_EPAD - _E,))])
    sk = lax.sort(keyp, is_stable=False)
    prev = jnp.concatenate([sk[:1] ^ jnp.uint32(1), sk[:-1]])
    w = (sk != prev).astype(jnp.float32)
    sidx = (sk >> 16).astype(jnp.int32)
    ridx = (sk & jnp.uint32(0xFFFF)).astype(jnp.int32)

    pad = ((0, _NPAD - _N), (0, 0))
    wpT = jnp.pad(world_pos, pad).T
    pwpT = jnp.pad(prev_world_pos, pad).T
    ntT = jnp.pad(node_type, pad).T
    b1col = b1[:, None]
    b2col = b2[:, None]

    deg2 = _deg_call(sidx, ridx, w).reshape(_NC, _NPAD)
    z1, dis = _prep_call(wpT, pwpT, ntT, deg2, W1, b1col)

    g1 = _from_planes(_msg_call(sidx, ridx, w, z1[0], z1[1], z1[2]))
    z2 = _mid_call(g1, dis, z1, W2, b2col)

    g2 = _from_planes(_msg_call(sidx, ridx, w, z2[0], z2[1], z2[2]))
    h2T = _fin_call(g2, dis, z2)

    h = h2T[:, :_N].T
    return jnp.where(is_training != 0, h, 2.0 * world_pos + h - prev_world_pos)
